# Initial kernel scaffold; baseline (speedup 1.0000x reference)
import jax
import jax.numpy as jnp
from jax.experimental import pallas as pl


def kernel(h_inv_tokens, h_dep_tokens, W_inv, b_inv, W_dep, b_dep, cb_inv, cb_dep, W_dec, b_dec):
    raise NotImplementedError("write your pallas kernel here")


# speedup vs baseline: 1.1270x; 1.1270x over previous
import functools

import jax
import jax.numpy as jnp
from jax import lax
from jax.experimental import pallas as pl
from jax.experimental.pallas import tpu as pltpu
from jax.experimental.pallas import tpu_sc as plsc

_L = 768
_D = 256
_K = 8192
_Tb = 256
_Kb = 2048
_NKB = _K // _Kb
_EPS = 1e-8


def _vq_body(tok_ref, w_ref, b_ref, cb_ref, idx_ref, s_ref,
             cbn_ref, xn_ref, mx_ref, am_ref):
    i = pl.program_id(0)
    k = pl.program_id(1)

    @pl.when(i == 0)
    def _():
        cb = cb_ref[pl.ds(k * _Kb, _Kb), :]
        nrm = jnp.sqrt(jnp.sum(cb * cb, axis=1, keepdims=True))
        cbn_ref[pl.ds(k * _Kb, _Kb), :] = cb / (nrm + _EPS)

    @pl.when(k == 0)
    def _():
        h = jnp.dot(tok_ref[...], w_ref[...],
                    preferred_element_type=jnp.float32) + b_ref[...]
        nrm = jnp.sqrt(jnp.sum(h * h, axis=1, keepdims=True))
        xn_ref[...] = h / (nrm + _EPS)

    sim = lax.dot_general(
        xn_ref[...], cbn_ref[pl.ds(k * _Kb, _Kb), :],
        (((1,), (1,)), ((), ())), preferred_element_type=jnp.float32)

    m = jnp.max(sim, axis=1, keepdims=True)
    iota = lax.broadcasted_iota(jnp.int32, sim.shape, 1) + k * _Kb
    amx = jnp.min(jnp.where(sim >= m, iota, _K), axis=1, keepdims=True)

    @pl.when(k == 0)
    def _():
        mx_ref[...] = jnp.full_like(mx_ref, -jnp.inf)
        am_ref[...] = jnp.zeros_like(am_ref)

    better = m > mx_ref[...]
    am_ref[...] = jnp.where(better, amx, am_ref[...])
    mx_ref[...] = jnp.where(better, m, mx_ref[...])

    @pl.when(k == _NKB - 1)
    def _():
        idx_ref[...] = am_ref[...].reshape(idx_ref.shape)
        tot = jnp.sum(mx_ref[...]).reshape(1, 1)

        @pl.when(i == 0)
        def _():
            s_ref[...] = tot

        @pl.when(i != 0)
        def _():
            s_ref[...] = s_ref[...] + tot


def _vq_head(tokens, W, b, cb):
    n = tokens.shape[0]
    grid_i = n // _Tb
    idx3, s = pl.pallas_call(
        _vq_body,
        grid=(grid_i, _NKB),
        in_specs=[
            pl.BlockSpec((_Tb, _L), lambda i, k: (i, 0)),
            pl.BlockSpec((_L, _D), lambda i, k: (0, 0)),
            pl.BlockSpec((1, _D), lambda i, k: (0, 0)),
            pl.BlockSpec((_K, _D), lambda i, k: (0, 0)),
        ],
        out_specs=[
            pl.BlockSpec((1, 1, _Tb), lambda i, k: (i, 0, 0)),
            pl.BlockSpec((1, 1), lambda i, k: (0, 0)),
        ],
        out_shape=[
            jax.ShapeDtypeStruct((grid_i, 1, _Tb), jnp.int32),
            jax.ShapeDtypeStruct((1, 1), jnp.float32),
        ],
        scratch_shapes=[
            pltpu.VMEM((_K, _D), jnp.float32),
            pltpu.VMEM((_Tb, _D), jnp.float32),
            pltpu.VMEM((_Tb, 1), jnp.float32),
            pltpu.VMEM((_Tb, 1), jnp.int32),
        ],
    )(tokens, W, b.reshape(1, _D), cb)
    return idx3.reshape(-1), s[0, 0]


def _sc_gather(cb, idx):
    info = plsc.get_sparse_core_info()
    nw = info.num_cores * info.num_subcores
    n = idx.shape[0]
    bpw = n // nw
    mesh = plsc.VectorSubcoreMesh(core_axis_name="c", subcore_axis_name="s")

    def body(cb_hbm, idx_hbm, out_hbm, idx_v, rows_v, sem):
        wid = lax.axis_index("s") * info.num_cores + lax.axis_index("c")
        base = wid * bpw
        pltpu.sync_copy(idx_hbm.at[pl.ds(base, bpw)], idx_v)
        pltpu.async_copy(cb_hbm.at[idx_v], rows_v, sem).wait()
        pltpu.sync_copy(rows_v, out_hbm.at[pl.ds(base, bpw)])

    return pl.kernel(
        body, mesh=mesh,
        out_type=jax.ShapeDtypeStruct((n, _D), jnp.float32),
        scratch_types=[
            pltpu.VMEM((bpw,), jnp.int32),
            pltpu.VMEM((bpw, _D), jnp.float32),
            pltpu.SemaphoreType.DMA,
        ],
    )(cb, idx)


def _dec_body(qi_ref, qd_ref, wd_ref, bd_ref, out_ref):
    qi = qi_ref[...]
    qi = qi / (jnp.sqrt(jnp.sum(qi * qi, axis=1, keepdims=True)) + _EPS)
    qd = qd_ref[...]
    qd = qd / (jnp.sqrt(jnp.sum(qd * qd, axis=1, keepdims=True)) + _EPS)
    acc = jnp.dot(qi, wd_ref[0:_D, :], preferred_element_type=jnp.float32)
    acc = acc + jnp.dot(qd, wd_ref[_D:2 * _D, :],
                        preferred_element_type=jnp.float32)
    out_ref[...] = acc + bd_ref[...]


def _decoder(q_inv, q_dep, W_dec, b_dec):
    n = q_inv.shape[0]
    blk = 512
    return pl.pallas_call(
        _dec_body,
        grid=(n // blk,),
        in_specs=[
            pl.BlockSpec((blk, _D), lambda i: (i, 0)),
            pl.BlockSpec((blk, _D), lambda i: (i, 0)),
            pl.BlockSpec((2 * _D, _L), lambda i: (0, 0)),
            pl.BlockSpec((1, _L), lambda i: (0, 0)),
        ],
        out_specs=pl.BlockSpec((blk, _L), lambda i: (i, 0)),
        out_shape=jax.ShapeDtypeStruct((n, _L), jnp.float32),
    )(q_inv, q_dep, W_dec, b_dec.reshape(1, _L))


def kernel(h_inv_tokens, h_dep_tokens, W_inv, b_inv, W_dep, b_dep,
           cb_inv, cb_dep, W_dec, b_dec):
    beta = 0.25
    B, T, L = h_inv_tokens.shape
    n = B * T
    ti = h_inv_tokens.reshape(n, L)
    td = h_dep_tokens.reshape(n, L)

    idx_i, s_i = _vq_head(ti, W_inv, b_inv, cb_inv)
    idx_d, s_d = _vq_head(td, W_dep, b_dep, cb_dep)

    q_i = _sc_gather(cb_inv, idx_i)
    q_d = _sc_gather(cb_dep, idx_d)

    z = _decoder(q_i, q_d, W_dec, b_dec).reshape(B, T, L)

    scale = beta * 2.0 / (n * _D)
    loss_i = scale * (n - s_i)
    loss_d = scale * (n - s_d)
    return z, loss_i, loss_d, idx_i.reshape(B, T), idx_d.reshape(B, T)


# speedup vs baseline: 1.2144x; 1.0776x over previous
import functools

import jax
import jax.numpy as jnp
from jax import lax
from jax.experimental import pallas as pl
from jax.experimental.pallas import tpu as pltpu
from jax.experimental.pallas import tpu_sc as plsc

_L = 768
_D = 256
_K = 8192
_Tb = 256
_Kb = 2048
_NKB = _K // _Kb
_EPS = 1e-8


def _vq_body(tok_ref, w_ref, b_ref, cb_ref, idx_ref, s_ref,
             cbn_ref, xn_ref, mx_ref, am_ref, iota_ref):
    i = pl.program_id(0)
    k = pl.program_id(1)

    @pl.when(jnp.logical_and(i == 0, k == 0))
    def _():
        iota_ref[...] = lax.broadcasted_iota(
            jnp.int32, (_Tb, _Kb), 1).astype(jnp.float32)

    @pl.when(i == 0)
    def _():
        cb = cb_ref[pl.ds(k * _Kb, _Kb), :]
        nrm = jnp.sqrt(jnp.sum(cb * cb, axis=1, keepdims=True))
        cbn_ref[pl.ds(k * _Kb, _Kb), :] = cb / (nrm + _EPS)

    @pl.when(k == 0)
    def _():
        h = jnp.dot(tok_ref[...], w_ref[...],
                    preferred_element_type=jnp.float32) + b_ref[...]
        nrm = jnp.sqrt(jnp.sum(h * h, axis=1, keepdims=True))
        xn_ref[...] = h / (nrm + _EPS)

    sim = lax.dot_general(
        xn_ref[...], cbn_ref[pl.ds(k * _Kb, _Kb), :],
        (((1,), (1,)), ((), ())), preferred_element_type=jnp.float32)

    m = jnp.max(sim, axis=1, keepdims=True)
    aml = jnp.min(jnp.where(sim >= m, iota_ref[...], float(_Kb)),
                  axis=1, keepdims=True)
    amx = aml.astype(jnp.int32) + k * _Kb

    @pl.when(k == 0)
    def _():
        mx_ref[...] = jnp.full_like(mx_ref, -jnp.inf)
        am_ref[...] = jnp.zeros_like(am_ref)

    better = m > mx_ref[...]
    am_ref[...] = jnp.where(better, amx, am_ref[...])
    mx_ref[...] = jnp.where(better, m, mx_ref[...])

    @pl.when(k == _NKB - 1)
    def _():
        idx_ref[...] = am_ref[...].reshape(idx_ref.shape)
        tot = jnp.sum(mx_ref[...]).reshape(1, 1)

        @pl.when(i == 0)
        def _():
            s_ref[...] = tot

        @pl.when(i != 0)
        def _():
            s_ref[...] = s_ref[...] + tot


def _vq_head(tokens, W, b, cb):
    n = tokens.shape[0]
    grid_i = n // _Tb
    idx3, s = pl.pallas_call(
        _vq_body,
        grid=(grid_i, _NKB),
        in_specs=[
            pl.BlockSpec((_Tb, _L), lambda i, k: (i, 0)),
            pl.BlockSpec((_L, _D), lambda i, k: (0, 0)),
            pl.BlockSpec((1, _D), lambda i, k: (0, 0)),
            pl.BlockSpec((_K, _D), lambda i, k: (0, 0)),
        ],
        out_specs=[
            pl.BlockSpec((1, 1, _Tb), lambda i, k: (i, 0, 0)),
            pl.BlockSpec((1, 1), lambda i, k: (0, 0)),
        ],
        out_shape=[
            jax.ShapeDtypeStruct((grid_i, 1, _Tb), jnp.int32),
            jax.ShapeDtypeStruct((1, 1), jnp.float32),
        ],
        scratch_shapes=[
            pltpu.VMEM((_K, _D), jnp.float32),
            pltpu.VMEM((_Tb, _D), jnp.float32),
            pltpu.VMEM((_Tb, 1), jnp.float32),
            pltpu.VMEM((_Tb, 1), jnp.int32),
            pltpu.VMEM((_Tb, _Kb), jnp.float32),
        ],
    )(tokens, W, b.reshape(1, _D), cb)
    return idx3.reshape(-1), s[0, 0]


def _sc_gather(cb, idx):
    info = plsc.get_sparse_core_info()
    nw = info.num_cores * info.num_subcores
    n = idx.shape[0]
    bpw = n // nw
    mesh = plsc.VectorSubcoreMesh(core_axis_name="c", subcore_axis_name="s")

    def body(cb_hbm, idx_hbm, out_hbm, idx_v, rows_v, sem):
        wid = lax.axis_index("s") * info.num_cores + lax.axis_index("c")
        base = wid * bpw
        pltpu.sync_copy(idx_hbm.at[pl.ds(base, bpw)], idx_v)
        pltpu.async_copy(cb_hbm.at[idx_v], rows_v, sem).wait()
        pltpu.sync_copy(rows_v, out_hbm.at[pl.ds(base, bpw)])

    return pl.kernel(
        body, mesh=mesh,
        out_type=jax.ShapeDtypeStruct((n, _D), jnp.float32),
        scratch_types=[
            pltpu.VMEM((bpw,), jnp.int32),
            pltpu.VMEM((bpw, _D), jnp.float32),
            pltpu.SemaphoreType.DMA,
        ],
    )(cb, idx)


def _dec_body(qi_ref, qd_ref, wd_ref, bd_ref, out_ref):
    qi = qi_ref[...]
    qi = qi / (jnp.sqrt(jnp.sum(qi * qi, axis=1, keepdims=True)) + _EPS)
    qd = qd_ref[...]
    qd = qd / (jnp.sqrt(jnp.sum(qd * qd, axis=1, keepdims=True)) + _EPS)
    acc = jnp.dot(qi, wd_ref[0:_D, :], preferred_element_type=jnp.float32)
    acc = acc + jnp.dot(qd, wd_ref[_D:2 * _D, :],
                        preferred_element_type=jnp.float32)
    out_ref[...] = acc + bd_ref[...]


def _decoder(q_inv, q_dep, W_dec, b_dec):
    n = q_inv.shape[0]
    blk = 512
    return pl.pallas_call(
        _dec_body,
        grid=(n // blk,),
        in_specs=[
            pl.BlockSpec((blk, _D), lambda i: (i, 0)),
            pl.BlockSpec((blk, _D), lambda i: (i, 0)),
            pl.BlockSpec((2 * _D, _L), lambda i: (0, 0)),
            pl.BlockSpec((1, _L), lambda i: (0, 0)),
        ],
        out_specs=pl.BlockSpec((blk, _L), lambda i: (i, 0)),
        out_shape=jax.ShapeDtypeStruct((n, _L), jnp.float32),
    )(q_inv, q_dep, W_dec, b_dec.reshape(1, _L))


def kernel(h_inv_tokens, h_dep_tokens, W_inv, b_inv, W_dep, b_dep,
           cb_inv, cb_dep, W_dec, b_dec):
    beta = 0.25
    B, T, L = h_inv_tokens.shape
    n = B * T
    ti = h_inv_tokens.reshape(n, L)
    td = h_dep_tokens.reshape(n, L)

    idx_i, s_i = _vq_head(ti, W_inv, b_inv, cb_inv)
    q_i = _sc_gather(cb_inv, idx_i)
    idx_d, s_d = _vq_head(td, W_dep, b_dep, cb_dep)
    q_d = _sc_gather(cb_dep, idx_d)

    z = _decoder(q_i, q_d, W_dec, b_dec).reshape(B, T, L)

    scale = beta * 2.0 / (n * _D)
    loss_i = scale * (n - s_i)
    loss_d = scale * (n - s_d)
    return z, loss_i, loss_d, idx_i.reshape(B, T), idx_d.reshape(B, T)


# speedup vs baseline: 1.6493x; 1.3581x over previous
import functools

import jax
import jax.numpy as jnp
from jax import lax
from jax.experimental import pallas as pl
from jax.experimental.pallas import tpu as pltpu
from jax.experimental.pallas import tpu_sc as plsc

_L = 768
_D = 256
_K = 8192
_Tb = 256
_Kb = 2048
_NKB = _K // _Kb
_EPS = 1e-8


def _vq_body(tok_ref, w_ref, b_ref, cb_ref, idx_ref, s_ref,
             cbn_ref, iota_ref):
    i = pl.program_id(0)

    @pl.when(i == 0)
    def _():
        iota_ref[...] = lax.broadcasted_iota(
            jnp.int32, (_Tb, _K), 1).astype(jnp.float32)
        cb = cb_ref[...]
        nrm = jnp.sqrt(jnp.sum(cb * cb, axis=1, keepdims=True))
        cbn_ref[...] = cb / (nrm + _EPS)

    h = jnp.dot(tok_ref[...], w_ref[...],
                preferred_element_type=jnp.float32) + b_ref[...]
    nrm = jnp.sqrt(jnp.sum(h * h, axis=1, keepdims=True))
    xn = h / (nrm + _EPS)

    sim = lax.dot_general(
        xn, cbn_ref[...],
        (((1,), (1,)), ((), ())), preferred_element_type=jnp.float32)

    m = jnp.max(sim, axis=1, keepdims=True)
    aml = jnp.min(jnp.where(sim >= m, iota_ref[...], float(_K)),
                  axis=1, keepdims=True)
    idx_ref[...] = aml.astype(jnp.int32).reshape(idx_ref.shape)

    tot = jnp.sum(m).reshape(1, 1)

    @pl.when(i == 0)
    def _():
        s_ref[...] = tot

    @pl.when(i != 0)
    def _():
        s_ref[...] = s_ref[...] + tot


def _vq_head(tokens, W, b, cb):
    n = tokens.shape[0]
    grid_i = n // _Tb
    idx3, s = pl.pallas_call(
        _vq_body,
        grid=(grid_i,),
        in_specs=[
            pl.BlockSpec((_Tb, _L), lambda i: (i, 0)),
            pl.BlockSpec((_L, _D), lambda i: (0, 0)),
            pl.BlockSpec((1, _D), lambda i: (0, 0)),
            pl.BlockSpec((_K, _D), lambda i: (0, 0)),
        ],
        out_specs=[
            pl.BlockSpec((1, 1, _Tb), lambda i: (i, 0, 0)),
            pl.BlockSpec((1, 1), lambda i: (0, 0)),
        ],
        out_shape=[
            jax.ShapeDtypeStruct((grid_i, 1, _Tb), jnp.int32),
            jax.ShapeDtypeStruct((1, 1), jnp.float32),
        ],
        scratch_shapes=[
            pltpu.VMEM((_K, _D), jnp.float32),
            pltpu.VMEM((_Tb, _K), jnp.float32),
        ],
    )(tokens, W, b.reshape(1, _D), cb)
    return idx3.reshape(-1), s[0, 0]


def _sc_gather(cb, idx):
    info = plsc.get_sparse_core_info()
    nw = info.num_cores * info.num_subcores
    n = idx.shape[0]
    bpw = n // nw
    mesh = plsc.VectorSubcoreMesh(core_axis_name="c", subcore_axis_name="s")

    def body(cb_hbm, idx_hbm, out_hbm, idx_v, rows_v, sem):
        wid = lax.axis_index("s") * info.num_cores + lax.axis_index("c")
        base = wid * bpw
        pltpu.sync_copy(idx_hbm.at[pl.ds(base, bpw)], idx_v)
        pltpu.async_copy(cb_hbm.at[idx_v], rows_v, sem).wait()
        pltpu.sync_copy(rows_v, out_hbm.at[pl.ds(base, bpw)])

    return pl.kernel(
        body, mesh=mesh,
        out_type=jax.ShapeDtypeStruct((n, _D), jnp.float32),
        scratch_types=[
            pltpu.VMEM((bpw,), jnp.int32),
            pltpu.VMEM((bpw, _D), jnp.float32),
            pltpu.SemaphoreType.DMA,
        ],
    )(cb, idx)


def _dec_body(qi_ref, qd_ref, wd_ref, bd_ref, out_ref):
    qi = qi_ref[...]
    qi = qi / (jnp.sqrt(jnp.sum(qi * qi, axis=1, keepdims=True)) + _EPS)
    qd = qd_ref[...]
    qd = qd / (jnp.sqrt(jnp.sum(qd * qd, axis=1, keepdims=True)) + _EPS)
    acc = jnp.dot(qi, wd_ref[0:_D, :], preferred_element_type=jnp.float32)
    acc = acc + jnp.dot(qd, wd_ref[_D:2 * _D, :],
                        preferred_element_type=jnp.float32)
    out_ref[...] = acc + bd_ref[...]


def _decoder(q_inv, q_dep, W_dec, b_dec):
    n = q_inv.shape[0]
    blk = 512
    return pl.pallas_call(
        _dec_body,
        grid=(n // blk,),
        in_specs=[
            pl.BlockSpec((blk, _D), lambda i: (i, 0)),
            pl.BlockSpec((blk, _D), lambda i: (i, 0)),
            pl.BlockSpec((2 * _D, _L), lambda i: (0, 0)),
            pl.BlockSpec((1, _L), lambda i: (0, 0)),
        ],
        out_specs=pl.BlockSpec((blk, _L), lambda i: (i, 0)),
        out_shape=jax.ShapeDtypeStruct((n, _L), jnp.float32),
    )(q_inv, q_dep, W_dec, b_dec.reshape(1, _L))


def kernel(h_inv_tokens, h_dep_tokens, W_inv, b_inv, W_dep, b_dep,
           cb_inv, cb_dep, W_dec, b_dec):
    beta = 0.25
    B, T, L = h_inv_tokens.shape
    n = B * T
    ti = h_inv_tokens.reshape(n, L)
    td = h_dep_tokens.reshape(n, L)

    idx_i, s_i = _vq_head(ti, W_inv, b_inv, cb_inv)
    q_i = _sc_gather(cb_inv, idx_i)
    idx_d, s_d = _vq_head(td, W_dep, b_dep, cb_dep)
    q_d = _sc_gather(cb_dep, idx_d)

    z = _decoder(q_i, q_d, W_dec, b_dec).reshape(B, T, L)

    scale = beta * 2.0 / (n * _D)
    loss_i = scale * (n - s_i)
    loss_d = scale * (n - s_d)
    return z, loss_i, loss_d, idx_i.reshape(B, T), idx_d.reshape(B, T)
